# baseline (speedup 1.0000x reference)
import functools

import jax
import jax.numpy as jnp
from jax import lax
from jax.experimental import pallas as pl
from jax.experimental.pallas import tpu as pltpu
from jax.experimental.pallas import tpu_sc as plsc

N = 10000
E = 320000
DF = 128
DH = 64

NC = 2
NS = 16
NW = NC * NS

NPAD = 10240
PAD_ROW = N
CHUNK = 128
NCH = 81
EPT = NCH * CHUNK
EPAD = NW * EPT
RPS = NPAD // NS

_mesh = plsc.VectorSubcoreMesh(
    core_axis_name="c", subcore_axis_name="s", num_cores=NC, num_subcores=NS)


def _wid():
    return lax.axis_index("c") * NS + lax.axis_index("s")


@functools.partial(
    pl.kernel,
    out_type=jax.ShapeDtypeStruct((NW, NPAD), jnp.float32),
    mesh=_mesh,
    compiler_params=pltpu.CompilerParams(needs_layout_passes=False),
    scratch_types=[
        pltpu.VMEM((EPT,), jnp.int32),
        pltpu.VMEM((NPAD,), jnp.float32),
    ],
)
def _deg_kernel(dst_hbm, cnt_out, dst_v, cnt_v):
    w = _wid()
    pltpu.sync_copy(dst_hbm.at[w], dst_v)
    zero16 = jnp.zeros((16,), jnp.float32)
    ones16 = jnp.ones((16,), jnp.float32)

    def zbody(i, carry):
        cnt_v[pl.ds(i * 16, 16)] = zero16
        return carry

    lax.fori_loop(0, NPAD // 16, zbody, 0)

    def ebody(i, carry):
        idx = dst_v[pl.ds(i * 16, 16)]
        plsc.addupdate_scatter(cnt_v, [idx], ones16)
        return carry

    lax.fori_loop(0, EPT // 16, ebody, 0)
    pltpu.sync_copy(cnt_v, cnt_out.at[w])


@functools.partial(
    pl.kernel,
    out_type=jax.ShapeDtypeStruct((NC, NPAD, DH), jnp.float32),
    mesh=_mesh,
    compiler_params=pltpu.CompilerParams(use_tc_tiling_on_sc=False),
    scratch_types=[
        pltpu.VMEM((NCH, CHUNK), jnp.int32),
        pltpu.VMEM((NCH, CHUNK), jnp.int32),
        pltpu.VMEM((CHUNK, DH), jnp.float32),
        pltpu.VMEM((CHUNK, DH), jnp.float32),
        pltpu.VMEM((CHUNK, DH), jnp.float32),
        pltpu.VMEM_SHARED((NPAD, DH), jnp.float32),
        pltpu.VMEM_SHARED((NPAD, DH), jnp.float32),
        pltpu.SemaphoreType.DMA,
        pltpu.SemaphoreType.DMA,
        pltpu.SemaphoreType.DMA,
        pltpu.SemaphoreType.DMA,
        pltpu.SemaphoreType.DMA,
        pltpu.SemaphoreType.DMA,
    ],
)
def _agg_kernel(table_hbm, src_hbm, dst_hbm, parts_out,
                src_v, dst_v, gbuf, gbuf_b, gbuf_c, table_sh, acc_sh,
                sem, sem_b, sem_c, ssem, ssem_b, ssem_c):
    c = lax.axis_index("c")
    s = lax.axis_index("s")
    w = c * NS + s
    pltpu.sync_copy(src_hbm.at[w], src_v)
    pltpu.sync_copy(dst_hbm.at[w], dst_v)

    base0 = s * RPS
    pltpu.sync_copy(table_hbm.at[pl.ds(base0, RPS)], table_sh.at[pl.ds(base0, RPS)])

    zero16 = jnp.zeros((16,), jnp.float32)

    def zbody(i, carry):
        for k in range(DH // 16):
            gbuf[i, pl.ds(k * 16, 16)] = zero16
        return carry

    lax.fori_loop(0, CHUNK, zbody, 0)
    for t in range(RPS // CHUNK):
        pltpu.sync_copy(gbuf, acc_sh.at[pl.ds(s * RPS + t * CHUNK, CHUNK)])
    plsc.subcore_barrier()

    GB = (gbuf, gbuf_b, gbuf_c)
    GS = (sem, sem_b, sem_c)
    SS = (ssem, ssem_b, ssem_c)
    pltpu.async_copy(table_sh.at[src_v.at[0]], GB[0], GS[0])
    pltpu.async_copy(table_sh.at[src_v.at[1]], GB[1], GS[1])

    def ebody(i, carry):
        for k in range(3):
            j = 3 * i + k
            nb = (k + 2) % 3
            if k == 0:
                @pl.when(i > 0)
                def _():
                    pltpu.make_async_copy(
                        GB[nb], acc_sh.at[dst_v.at[j - 1]], SS[nb]).wait()
            else:
                pltpu.make_async_copy(
                    GB[nb], acc_sh.at[dst_v.at[j - 1]], SS[nb]).wait()

            @pl.when(j + 2 < NCH)
            def _():
                pltpu.async_copy(
                    table_sh.at[src_v.at[jnp.minimum(j + 2, NCH - 1)]],
                    GB[nb], GS[nb])

            pltpu.make_async_copy(table_sh.at[src_v.at[j]], GB[k], GS[k]).wait()
            pltpu.async_copy(GB[k], acc_sh.at[dst_v.at[j]], SS[k], add=True)
        return carry

    lax.fori_loop(0, NCH // 3, ebody, 0)
    pltpu.make_async_copy(
        GB[(NCH - 1) % 3], acc_sh.at[dst_v.at[NCH - 1]],
        SS[(NCH - 1) % 3]).wait()
    plsc.subcore_barrier()

    pltpu.sync_copy(acc_sh.at[pl.ds(base0, RPS)], parts_out.at[c, pl.ds(base0, RPS)])


BR = 512
GRID = NPAD // BR
BRV = 400
GRIDV = N // BRV


def _tcA1_body(x_ref, w_ref, h_ref):
    h = jnp.dot(x_ref[...], w_ref[...], preferred_element_type=jnp.float32)
    rows = lax.broadcasted_iota(jnp.int32, (BR,), 0) + pl.program_id(0) * BR
    h_ref[...] = jnp.where(rows[:, None] < N, h, 0.0)


def _tcA1(x, W1):
    return pl.pallas_call(
        _tcA1_body,
        grid=(GRID,),
        in_specs=[
            pl.BlockSpec((BR, DF), lambda i: (i, 0)),
            pl.BlockSpec((DF, DH), lambda i: (0, 0)),
        ],
        out_specs=pl.BlockSpec((BR, DH), lambda i: (i, 0)),
        out_shape=jax.ShapeDtypeStruct((NPAD, DH), jnp.float32),
    )(x, W1)


def _tcA2_body(h_ref, cnt_ref, hd_ref, d_ref):
    cnt = jnp.sum(cnt_ref[...], axis=0)
    dv = lax.rsqrt(cnt + 1.0)
    rows = lax.broadcasted_iota(jnp.int32, (BR,), 0) + pl.program_id(0) * BR
    dv = jnp.where(rows < N, dv, 0.0)
    hd_ref[...] = h_ref[...] * dv[:, None]
    d_ref[...] = dv[:, None]


def _tcA2(h, cnt):
    return pl.pallas_call(
        _tcA2_body,
        grid=(GRID,),
        in_specs=[
            pl.BlockSpec((BR, DH), lambda i: (i, 0)),
            pl.BlockSpec((NW, BR), lambda i: (0, i)),
        ],
        out_specs=[
            pl.BlockSpec((BR, DH), lambda i: (i, 0)),
            pl.BlockSpec((BR, 1), lambda i: (i, 0)),
        ],
        out_shape=[
            jax.ShapeDtypeStruct((NPAD, DH), jnp.float32),
            jax.ShapeDtypeStruct((NPAD, 1), jnp.float32),
        ],
    )(h, cnt)


def _tcB_body(p_ref, hd_ref, d_ref, b1_ref, w2_ref, emb_ref, gd_ref):
    dv = d_ref[...]
    emb = (p_ref[0] + p_ref[1] + hd_ref[...]) * dv + b1_ref[...]
    emb_ref[...] = emb
    r = jnp.maximum(emb, 0.0)
    g = jnp.dot(r, w2_ref[...], preferred_element_type=jnp.float32)
    gd_ref[...] = g * dv


def _tcB(p1, hd, d, b1, W2):
    return pl.pallas_call(
        _tcB_body,
        grid=(GRID,),
        in_specs=[
            pl.BlockSpec((NC, BR, DH), lambda i: (0, i, 0)),
            pl.BlockSpec((BR, DH), lambda i: (i, 0)),
            pl.BlockSpec((BR, 1), lambda i: (i, 0)),
            pl.BlockSpec((1, DH), lambda i: (0, 0)),
            pl.BlockSpec((DH, DH), lambda i: (0, 0)),
        ],
        out_specs=[
            pl.BlockSpec((BR, DH), lambda i: (i, 0)),
            pl.BlockSpec((BR, DH), lambda i: (i, 0)),
        ],
        out_shape=[
            jax.ShapeDtypeStruct((NPAD, DH), jnp.float32),
            jax.ShapeDtypeStruct((NPAD, DH), jnp.float32),
        ],
    )(p1, hd, d, b1, W2)


def _tcC_body(p_ref, gd_ref, d_ref, b2_ref, out_ref):
    logits = (p_ref[0] + p_ref[1] + gd_ref[...]) * d_ref[...] + b2_ref[...]
    m = jnp.max(logits, axis=1, keepdims=True)
    lse = jnp.log(jnp.sum(jnp.exp(logits - m), axis=1, keepdims=True)) + m
    out_ref[...] = logits - lse


def _tcC(p2, gd, d, b2):
    return pl.pallas_call(
        _tcC_body,
        grid=(GRID,),
        in_specs=[
            pl.BlockSpec((NC, BR, DH), lambda i: (0, i, 0)),
            pl.BlockSpec((BR, DH), lambda i: (i, 0)),
            pl.BlockSpec((BR, 1), lambda i: (i, 0)),
            pl.BlockSpec((1, DH), lambda i: (0, 0)),
        ],
        out_specs=pl.BlockSpec((BR, DH), lambda i: (i, 0)),
        out_shape=jax.ShapeDtypeStruct((NPAD, DH), jnp.float32),
    )(p2, gd, d, b2)


def kernel(x, edge_index, W1, b1, W2, b2):
    src = edge_index[0].astype(jnp.int32)
    dst = edge_index[1].astype(jnp.int32)
    padv = jnp.full((EPAD - E,), PAD_ROW, jnp.int32)
    srcp = jnp.concatenate([src, padv]).reshape(NW, NCH, CHUNK)
    dstp = jnp.concatenate([dst, padv]).reshape(NW, NCH, CHUNK)

    cnt = _deg_kernel(dstp.reshape(NW, EPT))
    h = _tcA1(x, W1)
    hd, d = _tcA2(h, cnt)
    p1 = _agg_kernel(hd, srcp, dstp)
    emb, gd = _tcB(p1, hd, d, b1[None, :], W2)
    p2 = _agg_kernel(gd, srcp, dstp)
    logp = _tcC(p2, gd, d, b2[None, :])
    return logp[:N], emb[:N]


# speedup vs baseline: 1.1079x; 1.1079x over previous
import functools

import jax
import jax.numpy as jnp
from jax import lax
from jax.experimental import pallas as pl
from jax.experimental.pallas import tpu as pltpu
from jax.experimental.pallas import tpu_sc as plsc

N = 10000
E = 320000
DF = 128
DH = 64

NC = 2
NS = 16
NW = NC * NS

NPAD = 10240
PAD_ROW = N
CHUNK = 128
NCH = 81
EPT = NCH * CHUNK
EPAD = NW * EPT
RPS = NPAD // NS

_mesh = plsc.VectorSubcoreMesh(
    core_axis_name="c", subcore_axis_name="s", num_cores=NC, num_subcores=NS)


def _wid():
    return lax.axis_index("c") * NS + lax.axis_index("s")


@functools.partial(
    pl.kernel,
    out_type=jax.ShapeDtypeStruct((NW, NPAD), jnp.float32),
    mesh=_mesh,
    compiler_params=pltpu.CompilerParams(needs_layout_passes=False),
    scratch_types=[
        pltpu.VMEM((NCH, CHUNK), jnp.int32),
        pltpu.VMEM((NPAD,), jnp.float32),
    ],
)
def _deg_kernel(dst_hbm, cnt_out, dst_v, cnt_v):
    w = _wid()
    pltpu.sync_copy(dst_hbm.at[:, w], dst_v)
    zero16 = jnp.zeros((16,), jnp.float32)
    ones16 = jnp.ones((16,), jnp.float32)

    def zbody(i, carry):
        cnt_v[pl.ds(i * 16, 16)] = zero16
        return carry

    lax.fori_loop(0, NPAD // 16, zbody, 0)

    def ebody(i, carry):
        idx = dst_v[i // 8, pl.ds((i % 8) * 16, 16)]
        plsc.addupdate_scatter(cnt_v, [idx], ones16)
        return carry

    lax.fori_loop(0, EPT // 16, ebody, 0)
    pltpu.sync_copy(cnt_v, cnt_out.at[w])


@functools.partial(
    pl.kernel,
    out_type=jax.ShapeDtypeStruct((NC, NPAD, DH), jnp.float32),
    mesh=_mesh,
    compiler_params=pltpu.CompilerParams(use_tc_tiling_on_sc=False),
    scratch_types=[
        pltpu.VMEM((NCH, CHUNK), jnp.int32),
        pltpu.VMEM((NCH, CHUNK), jnp.int32),
        pltpu.VMEM((CHUNK, DH), jnp.float32),
        pltpu.VMEM((CHUNK, DH), jnp.float32),
        pltpu.VMEM((CHUNK, DH), jnp.float32),
        pltpu.VMEM_SHARED((NPAD, DH), jnp.float32),
        pltpu.VMEM_SHARED((NPAD, DH), jnp.float32),
        pltpu.SemaphoreType.DMA,
        pltpu.SemaphoreType.DMA,
        pltpu.SemaphoreType.DMA,
        pltpu.SemaphoreType.DMA,
        pltpu.SemaphoreType.DMA,
        pltpu.SemaphoreType.DMA,
    ],
)
def _agg_kernel(table_hbm, src_hbm, dst_hbm, parts_out,
                src_v, dst_v, gbuf, gbuf_b, gbuf_c, table_sh, acc_sh,
                sem, sem_b, sem_c, ssem, ssem_b, ssem_c):
    c = lax.axis_index("c")
    s = lax.axis_index("s")
    w = c * NS + s
    pltpu.sync_copy(src_hbm.at[:, w], src_v)
    pltpu.sync_copy(dst_hbm.at[:, w], dst_v)

    base0 = s * RPS
    pltpu.sync_copy(table_hbm.at[pl.ds(base0, RPS)], table_sh.at[pl.ds(base0, RPS)])

    zero16 = jnp.zeros((16,), jnp.float32)

    def zbody(i, carry):
        for k in range(DH // 16):
            gbuf[i, pl.ds(k * 16, 16)] = zero16
        return carry

    lax.fori_loop(0, CHUNK, zbody, 0)
    for t in range(RPS // CHUNK):
        pltpu.sync_copy(gbuf, acc_sh.at[pl.ds(s * RPS + t * CHUNK, CHUNK)])
    plsc.subcore_barrier()

    GB = (gbuf, gbuf_b, gbuf_c)
    GS = (sem, sem_b, sem_c)
    SS = (ssem, ssem_b, ssem_c)
    pltpu.async_copy(table_sh.at[src_v.at[0]], GB[0], GS[0])
    pltpu.async_copy(table_sh.at[src_v.at[1]], GB[1], GS[1])

    def ebody(i, carry):
        for k in range(3):
            j = 3 * i + k
            nb = (k + 2) % 3
            if k == 0:
                @pl.when(i > 0)
                def _():
                    pltpu.make_async_copy(
                        GB[nb], acc_sh.at[dst_v.at[j - 1]], SS[nb]).wait()
            else:
                pltpu.make_async_copy(
                    GB[nb], acc_sh.at[dst_v.at[j - 1]], SS[nb]).wait()

            @pl.when(j + 2 < NCH)
            def _():
                pltpu.async_copy(
                    table_sh.at[src_v.at[jnp.minimum(j + 2, NCH - 1)]],
                    GB[nb], GS[nb])

            pltpu.make_async_copy(table_sh.at[src_v.at[j]], GB[k], GS[k]).wait()
            pltpu.async_copy(GB[k], acc_sh.at[dst_v.at[j]], SS[k], add=True)
        return carry

    lax.fori_loop(0, NCH // 3, ebody, 0)
    pltpu.make_async_copy(
        GB[(NCH - 1) % 3], acc_sh.at[dst_v.at[NCH - 1]],
        SS[(NCH - 1) % 3]).wait()
    plsc.subcore_barrier()

    pltpu.sync_copy(acc_sh.at[pl.ds(base0, RPS)], parts_out.at[c, pl.ds(base0, RPS)])


BR = 2048
GRID = NPAD // BR
BRV = 400
GRIDV = N // BRV


def _tcA1_body(x_ref, w_ref, h_ref):
    h = jnp.dot(x_ref[...], w_ref[...], preferred_element_type=jnp.float32)
    rows = lax.broadcasted_iota(jnp.int32, (BR,), 0) + pl.program_id(0) * BR
    h_ref[...] = jnp.where(rows[:, None] < N, h, 0.0)


def _tcA1(x, W1):
    return pl.pallas_call(
        _tcA1_body,
        grid=(GRID,),
        in_specs=[
            pl.BlockSpec((BR, DF), lambda i: (i, 0)),
            pl.BlockSpec((DF, DH), lambda i: (0, 0)),
        ],
        out_specs=pl.BlockSpec((BR, DH), lambda i: (i, 0)),
        out_shape=jax.ShapeDtypeStruct((NPAD, DH), jnp.float32),
    )(x, W1)


def _tcA2_body(h_ref, cnt_ref, hd_ref, d_ref):
    cnt = jnp.sum(cnt_ref[...], axis=0)
    dv = lax.rsqrt(cnt + 1.0)
    rows = lax.broadcasted_iota(jnp.int32, (BR,), 0) + pl.program_id(0) * BR
    dv = jnp.where(rows < N, dv, 0.0)
    hd_ref[...] = h_ref[...] * dv[:, None]
    d_ref[...] = dv[:, None]


def _tcA2(h, cnt):
    return pl.pallas_call(
        _tcA2_body,
        grid=(GRID,),
        in_specs=[
            pl.BlockSpec((BR, DH), lambda i: (i, 0)),
            pl.BlockSpec((NW, BR), lambda i: (0, i)),
        ],
        out_specs=[
            pl.BlockSpec((BR, DH), lambda i: (i, 0)),
            pl.BlockSpec((BR, 1), lambda i: (i, 0)),
        ],
        out_shape=[
            jax.ShapeDtypeStruct((NPAD, DH), jnp.float32),
            jax.ShapeDtypeStruct((NPAD, 1), jnp.float32),
        ],
    )(h, cnt)


def _tcB_body(p_ref, hd_ref, d_ref, b1_ref, w2_ref, emb_ref, gd_ref):
    dv = d_ref[...]
    emb = (p_ref[0] + p_ref[1] + hd_ref[...]) * dv + b1_ref[...]
    emb_ref[...] = emb
    r = jnp.maximum(emb, 0.0)
    g = jnp.dot(r, w2_ref[...], preferred_element_type=jnp.float32)
    gd_ref[...] = g * dv


def _tcB(p1, hd, d, b1, W2):
    return pl.pallas_call(
        _tcB_body,
        grid=(GRID,),
        in_specs=[
            pl.BlockSpec((NC, BR, DH), lambda i: (0, i, 0)),
            pl.BlockSpec((BR, DH), lambda i: (i, 0)),
            pl.BlockSpec((BR, 1), lambda i: (i, 0)),
            pl.BlockSpec((1, DH), lambda i: (0, 0)),
            pl.BlockSpec((DH, DH), lambda i: (0, 0)),
        ],
        out_specs=[
            pl.BlockSpec((BR, DH), lambda i: (i, 0)),
            pl.BlockSpec((BR, DH), lambda i: (i, 0)),
        ],
        out_shape=[
            jax.ShapeDtypeStruct((NPAD, DH), jnp.float32),
            jax.ShapeDtypeStruct((NPAD, DH), jnp.float32),
        ],
    )(p1, hd, d, b1, W2)


def _tcC_body(p_ref, gd_ref, d_ref, b2_ref, out_ref):
    logits = (p_ref[0] + p_ref[1] + gd_ref[...]) * d_ref[...] + b2_ref[...]
    m = jnp.max(logits, axis=1, keepdims=True)
    lse = jnp.log(jnp.sum(jnp.exp(logits - m), axis=1, keepdims=True)) + m
    out_ref[...] = logits - lse


def _tcC(p2, gd, d, b2):
    return pl.pallas_call(
        _tcC_body,
        grid=(GRID,),
        in_specs=[
            pl.BlockSpec((NC, BR, DH), lambda i: (0, i, 0)),
            pl.BlockSpec((BR, DH), lambda i: (i, 0)),
            pl.BlockSpec((BR, 1), lambda i: (i, 0)),
            pl.BlockSpec((1, DH), lambda i: (0, 0)),
        ],
        out_specs=pl.BlockSpec((BR, DH), lambda i: (i, 0)),
        out_shape=jax.ShapeDtypeStruct((NPAD, DH), jnp.float32),
    )(p2, gd, d, b2)


def kernel(x, edge_index, W1, b1, W2, b2):
    src = edge_index[0].astype(jnp.int32)
    dst = edge_index[1].astype(jnp.int32)
    padv = jnp.full((EPAD - E,), PAD_ROW, jnp.int32)
    srcp = jnp.concatenate([src, padv]).reshape(NCH, NW, CHUNK)
    dstp = jnp.concatenate([dst, padv]).reshape(NCH, NW, CHUNK)

    cnt = _deg_kernel(dstp)
    h = _tcA1(x, W1)
    hd, d = _tcA2(h, cnt)
    p1 = _agg_kernel(hd, srcp, dstp)
    emb, gd = _tcB(p1, hd, d, b1[None, :], W2)
    p2 = _agg_kernel(gd, srcp, dstp)
    logp = _tcC(p2, gd, d, b2[None, :])
    return logp[:N], emb[:N]
